# baseline (speedup 1.0000x reference)
import jax
import jax.numpy as jnp
from jax.experimental import pallas as pl
from jax.experimental.pallas import tpu as pltpu


def _round_up(x, m):
    return ((x + m - 1) // m) * m


def _resident(shape):
    return pl.BlockSpec(shape, lambda i: (0, 0))


_CHUNKS = 2


def _mlp_softmax_kernel(x_ref, w1_ref, b1_ref, w2_ref, b2_ref, w3_ref,
                        b3_ref, out_ref):
    w1 = w1_ref[...].astype(jnp.bfloat16)
    w2 = w2_ref[...].astype(jnp.bfloat16)
    w3 = w3_ref[...].astype(jnp.bfloat16)
    b1 = b1_ref[...].astype(jnp.bfloat16)
    b2 = b2_ref[...].astype(jnp.bfloat16)
    b3 = b3_ref[...]
    rows = out_ref.shape[0] // _CHUNKS
    for c in range(_CHUNKS):
        sl = pl.ds(c * rows, rows)
        x = x_ref[sl, :].astype(jnp.bfloat16)
        h1 = jnp.dot(x, w1, preferred_element_type=jnp.float32)
        h1 = jnp.maximum(h1.astype(jnp.bfloat16) + b1, 0)
        h2 = jnp.dot(h1, w2, preferred_element_type=jnp.float32)
        h2 = jnp.maximum(h2.astype(jnp.bfloat16) + b2, 0)
        logits = jnp.dot(h2, w3, preferred_element_type=jnp.float32) + b3
        m = jnp.max(logits, axis=-1, keepdims=True)
        e = jnp.exp(logits - m)
        s = jnp.sum(e, axis=-1, keepdims=True)
        out_ref[sl, :] = e / s + 1e-8


def kernel(state, w1, b1, w2, b2, w3, b3):
    lead = state.shape[:-1]
    nin = state.shape[-1]
    x = state.reshape(-1, nin).astype(jnp.float32)
    b = x.shape[0]
    hid = w1.shape[1]
    nout = w3.shape[1]

    tb = min(1024, _round_up(max(b, 1), 8))
    if pl.cdiv(b, tb) == 1 and b >= 16:
        tb = _round_up(pl.cdiv(b, 2), 8)
    grid = (pl.cdiv(b, tb),)

    out = pl.pallas_call(
        _mlp_softmax_kernel,
        out_shape=jax.ShapeDtypeStruct((b, nout), jnp.float32),
        grid_spec=pltpu.PrefetchScalarGridSpec(
            num_scalar_prefetch=0,
            grid=grid,
            in_specs=[
                pl.BlockSpec((tb, nin), lambda i: (i, 0)),
                _resident((nin, hid)),
                _resident((1, hid)),
                _resident((hid, hid)),
                _resident((1, hid)),
                _resident((hid, nout)),
                _resident((1, nout)),
            ],
            out_specs=pl.BlockSpec((tb, nout), lambda i: (i, 0)),
        ),
        compiler_params=pltpu.CompilerParams(
            dimension_semantics=("arbitrary",)),
    )(x, w1, b1, w2, b2, w3, b3)

    return out.reshape(lead + (nout,))


# speedup vs baseline: 1.0537x; 1.0537x over previous
import jax
import jax.numpy as jnp
from jax.experimental import pallas as pl
from jax.experimental.pallas import tpu as pltpu


def _round_up(x, m):
    return ((x + m - 1) // m) * m


def _resident(shape):
    return pl.BlockSpec(shape, lambda i: (0, 0))


_CHUNKS = 2


def _mlp_softmax_kernel(x_ref, w1_ref, b1_ref, w2_hbm, b2_ref, w3_hbm,
                        b3_ref, out_ref, w2_vmem, w3_vmem, sem2, sem3):
    i = pl.program_id(0)

    @pl.when(i == 0)
    def _():
        pltpu.make_async_copy(w2_hbm, w2_vmem, sem2).start()
        pltpu.make_async_copy(w3_hbm, w3_vmem, sem3).start()

    w1 = w1_ref[...].astype(jnp.bfloat16)
    b1 = b1_ref[...].astype(jnp.bfloat16)
    rows = out_ref.shape[0] // _CHUNKS

    h1s = []
    for c in range(_CHUNKS):
        x = x_ref[pl.ds(c * rows, rows), :].astype(jnp.bfloat16)
        a1 = jnp.dot(x, w1, preferred_element_type=jnp.float32)
        h1s.append(jnp.maximum(a1.astype(jnp.bfloat16) + b1, 0))

    @pl.when(i == 0)
    def _():
        pltpu.make_async_copy(w2_hbm, w2_vmem, sem2).wait()
        pltpu.make_async_copy(w3_hbm, w3_vmem, sem3).wait()

    w2 = w2_vmem[...].astype(jnp.bfloat16)
    w3 = w3_vmem[...].astype(jnp.bfloat16)
    b2 = b2_ref[...].astype(jnp.bfloat16)
    b3 = b3_ref[...]
    for c in range(_CHUNKS):
        a2 = jnp.dot(h1s[c], w2, preferred_element_type=jnp.float32)
        h2 = jnp.maximum(a2.astype(jnp.bfloat16) + b2, 0)
        logits = jnp.dot(h2, w3, preferred_element_type=jnp.float32) + b3
        m = jnp.max(logits, axis=-1, keepdims=True)
        e = jnp.exp(logits - m)
        s = jnp.sum(e, axis=-1, keepdims=True)
        out_ref[pl.ds(c * rows, rows), :] = e / s + 1e-8


def kernel(state, w1, b1, w2, b2, w3, b3):
    lead = state.shape[:-1]
    nin = state.shape[-1]
    x = state.reshape(-1, nin).astype(jnp.float32)
    b = x.shape[0]
    hid = w1.shape[1]
    nout = w3.shape[1]

    tb = min(2048, _round_up(max(b, 1), 8))
    if pl.cdiv(b, tb) == 1 and b >= 16:
        tb = _round_up(pl.cdiv(b, 2), 8)
    grid = (pl.cdiv(b, tb),)

    out = pl.pallas_call(
        _mlp_softmax_kernel,
        out_shape=jax.ShapeDtypeStruct((b, nout), jnp.float32),
        grid_spec=pltpu.PrefetchScalarGridSpec(
            num_scalar_prefetch=0,
            grid=grid,
            in_specs=[
                pl.BlockSpec((tb, nin), lambda i: (i, 0)),
                _resident((nin, hid)),
                _resident((1, hid)),
                pl.BlockSpec(memory_space=pl.ANY),
                _resident((1, hid)),
                pl.BlockSpec(memory_space=pl.ANY),
                _resident((1, nout)),
            ],
            out_specs=pl.BlockSpec((tb, nout), lambda i: (i, 0)),
            scratch_shapes=[
                pltpu.VMEM((hid, hid), jnp.float32),
                pltpu.VMEM((hid, nout), jnp.float32),
                pltpu.SemaphoreType.DMA,
                pltpu.SemaphoreType.DMA,
            ],
        ),
        compiler_params=pltpu.CompilerParams(
            dimension_semantics=("arbitrary",)),
    )(x, w1, b1, w2, b2, w3, b3)

    return out.reshape(lead + (nout,))


# speedup vs baseline: 1.0629x; 1.0087x over previous
import jax
import jax.numpy as jnp
from jax.experimental import pallas as pl
from jax.experimental.pallas import tpu as pltpu


def _round_up(x, m):
    return ((x + m - 1) // m) * m


def _resident(shape):
    return pl.BlockSpec(shape, lambda i: (0, 0))


_CHUNKS = 2


def _mlp_softmax_kernel(x_ref, w1_ref, b1_ref, w2_hbm, b2_ref, w3_hbm,
                        b3_ref, out_ref, w2_vmem, w3_vmem, sem2, sem3):
    i = pl.program_id(0)

    @pl.when(i == 0)
    def _():
        pltpu.make_async_copy(w2_hbm, w2_vmem, sem2).start()
        pltpu.make_async_copy(w3_hbm, w3_vmem, sem3).start()

    w1 = w1_ref[...].astype(jnp.bfloat16)
    b1 = b1_ref[...]
    rows = out_ref.shape[0] // _CHUNKS

    h1s = []
    for c in range(_CHUNKS):
        x = x_ref[pl.ds(c * rows, rows), :].astype(jnp.bfloat16)
        a1 = jnp.dot(x, w1, preferred_element_type=jnp.float32)
        h1s.append(jnp.maximum(a1 + b1, 0.0).astype(jnp.bfloat16))

    @pl.when(i == 0)
    def _():
        pltpu.make_async_copy(w2_hbm, w2_vmem, sem2).wait()
        pltpu.make_async_copy(w3_hbm, w3_vmem, sem3).wait()

    w2 = w2_vmem[...].astype(jnp.bfloat16)
    w3 = w3_vmem[...].astype(jnp.bfloat16)
    b2 = b2_ref[...]
    b3 = b3_ref[...]
    for c in range(_CHUNKS):
        a2 = jnp.dot(h1s[c], w2, preferred_element_type=jnp.float32)
        h2 = jnp.maximum(a2 + b2, 0.0).astype(jnp.bfloat16)
        logits = jnp.dot(h2, w3, preferred_element_type=jnp.float32) + b3
        m = jnp.max(logits, axis=-1, keepdims=True)
        e = jnp.exp(logits - m)
        s = jnp.sum(e, axis=-1, keepdims=True)
        out_ref[pl.ds(c * rows, rows), :] = e / s + 1e-8


def kernel(state, w1, b1, w2, b2, w3, b3):
    lead = state.shape[:-1]
    nin = state.shape[-1]
    x = state.reshape(-1, nin).astype(jnp.float32)
    b = x.shape[0]
    hid = w1.shape[1]
    nout = w3.shape[1]

    tb = min(2048, _round_up(max(b, 1), 8))
    if pl.cdiv(b, tb) == 1 and b >= 16:
        tb = _round_up(pl.cdiv(b, 2), 8)
    grid = (pl.cdiv(b, tb),)

    out = pl.pallas_call(
        _mlp_softmax_kernel,
        out_shape=jax.ShapeDtypeStruct((b, nout), jnp.float32),
        grid_spec=pltpu.PrefetchScalarGridSpec(
            num_scalar_prefetch=0,
            grid=grid,
            in_specs=[
                pl.BlockSpec((tb, nin), lambda i: (i, 0)),
                _resident((nin, hid)),
                _resident((1, hid)),
                pl.BlockSpec(memory_space=pl.ANY),
                _resident((1, hid)),
                pl.BlockSpec(memory_space=pl.ANY),
                _resident((1, nout)),
            ],
            out_specs=pl.BlockSpec((tb, nout), lambda i: (i, 0)),
            scratch_shapes=[
                pltpu.VMEM((hid, hid), jnp.float32),
                pltpu.VMEM((hid, nout), jnp.float32),
                pltpu.SemaphoreType.DMA,
                pltpu.SemaphoreType.DMA,
            ],
        ),
        compiler_params=pltpu.CompilerParams(
            dimension_semantics=("arbitrary",)),
    )(x, w1, b1, w2, b2, w3, b3)

    return out.reshape(lead + (nout,))
